# Initial kernel scaffold; baseline (speedup 1.0000x reference)
import jax
import jax.numpy as jnp
from jax.experimental import pallas as pl


def kernel(hidden_states):
    raise NotImplementedError("write your pallas kernel here")


# speedup vs baseline: 2.5492x; 2.5492x over previous
import jax
import jax.numpy as jnp
from jax.experimental import pallas as pl

_SEQ = 4096
_HID = 1024
_BLK = 512


def _copy_zero_kernel(hid_ref, out_ref, attn_ref):
    out_ref[...] = hid_ref[...]
    attn_ref[...] = jnp.zeros_like(attn_ref)


def kernel(hidden_states):
    batch, seq, hid = hidden_states.shape
    h2 = hidden_states.reshape(seq, hid)
    out, attn = pl.pallas_call(
        _copy_zero_kernel,
        grid=(seq // _BLK,),
        in_specs=[pl.BlockSpec((_BLK, hid), lambda i: (i, 0))],
        out_specs=[
            pl.BlockSpec((_BLK, hid), lambda i: (i, 0)),
            pl.BlockSpec((_BLK, seq), lambda i: (i, 0)),
        ],
        out_shape=[
            jax.ShapeDtypeStruct((seq, hid), hidden_states.dtype),
            jax.ShapeDtypeStruct((seq, seq), hidden_states.dtype),
        ],
    )(h2)
    return (out.reshape(batch, seq, hid), attn.reshape(batch, seq, seq))


# speedup vs baseline: 2.5701x; 1.0082x over previous
import jax
import jax.numpy as jnp
from jax.experimental import pallas as pl
from jax.experimental.pallas import tpu as pltpu

_SEQ = 4096
_HID = 1024
_BLK = 512


def _copy_zero_kernel(hid_ref, out_ref, attn_ref):
    out_ref[...] = hid_ref[...]
    attn_ref[...] = jnp.zeros_like(attn_ref)


def kernel(hidden_states):
    batch, seq, hid = hidden_states.shape
    h2 = hidden_states.reshape(seq, hid)
    out, attn = pl.pallas_call(
        _copy_zero_kernel,
        grid=(seq // _BLK,),
        in_specs=[pl.BlockSpec((_BLK, hid), lambda i: (i, 0))],
        out_specs=[
            pl.BlockSpec((_BLK, hid), lambda i: (i, 0)),
            pl.BlockSpec((_BLK, seq), lambda i: (i, 0)),
        ],
        out_shape=[
            jax.ShapeDtypeStruct((seq, hid), hidden_states.dtype),
            jax.ShapeDtypeStruct((seq, seq), hidden_states.dtype),
        ],
        compiler_params=pltpu.CompilerParams(
            dimension_semantics=("parallel",),
        ),
    )(h2)
    return (out.reshape(batch, seq, hid), attn.reshape(batch, seq, seq))
